# baseline (speedup 1.0000x reference)
import functools

import jax
import jax.numpy as jnp
from jax import lax
from jax.experimental import pallas as pl
from jax.experimental.pallas import tpu as pltpu
from jax.experimental.pallas import tpu_sc as plsc

FIELD0 = 1000000
BATCH = 16384
D = 16
NF = 2 * D

_NW = 32
_COLS_PER_W = BATCH // _NW
_NG = _COLS_PER_W // 128
_CC = _COLS_PER_W // 16

_FOFF = tuple((c // 8) * 16000000 + (c % 8) * 128 for c in range(D))


def _gather_body(tab_hbm, x_hbm, out_hbm, xv, ebuf, rows_v, sem0, sem1):
    wid = lax.axis_index("s") * 2 + lax.axis_index("c")
    base = wid * 2 * _COLS_PER_W
    pltpu.sync_copy(x_hbm.at[pl.ds(base, 2 * _COLS_PER_W)], xv)
    sems = (sem0, sem1)

    def drain_and_flush(g):
        pltpu.make_async_copy(
            out_hbm.at[:, _NG * wid + g], rows_v.at[:, g], sems[g % 2]
        ).wait()
        pltpu.sync_copy(rows_v.at[:, g], out_hbm.at[:, _NG * wid + g])

    for g in range(_NG):
        def build(part, _, g=g):
            ru = xv[pl.ds(256 * g + 16 * part, 16)]
            ri = xv[pl.ds(256 * g + 128 + 16 * part, 16)] + FIELD0
            rpu = lax.shift_right_logical(ru, 7) * 1024 + (ru & 127)
            rpi = lax.shift_right_logical(ri, 7) * 1024 + (ri & 127)
            col = 128 * g + 16 * part
            for c in range(D):
                ebuf[c, pl.ds(col, 16)] = rpu + _FOFF[c]
                ebuf[D + c, pl.ds(col, 16)] = rpi + _FOFF[c]
            return _

        lax.fori_loop(0, 8, build, None)
        for f in range(NF):
            pltpu.async_copy(
                tab_hbm.at[ebuf.at[f, pl.ds(128 * g, 128)]],
                rows_v.at[f // 8, g, f % 8],
                sems[g % 2],
            )
        if g >= 1:
            drain_and_flush(g - 1)
    drain_and_flush(_NG - 1)


@functools.cache
def _make_gather():
    return functools.partial(
        pl.kernel,
        mesh=plsc.VectorSubcoreMesh(core_axis_name="c", subcore_axis_name="s"),
        out_type=jax.ShapeDtypeStruct((NF // 8, BATCH // 128, 8, 128), jnp.float32),
        scratch_types=[
            pltpu.VMEM((2 * _COLS_PER_W,), jnp.int32),
            pltpu.VMEM((NF, _COLS_PER_W), jnp.int32),
            pltpu.VMEM((NF // 8, _NG, 8, 128), jnp.float32),
            pltpu.SemaphoreType.DMA,
            pltpu.SemaphoreType.DMA,
        ],
        compiler_params=pltpu.CompilerParams(use_tc_tiling_on_sc=False),
    )(_gather_body)


def _mlp_body(et_ref, w0, b0, g0, be0, w1, b1, g1, be1, w2, b2, g2, be2,
              fcw, fcb, out_ref):
    et = et_ref[...]
    gmf = et[:D, :] * et[D:, :]
    h = et
    ones = jnp.full((BATCH, 1), 1.0 / BATCH, dtype=jnp.float32)
    for (w, g, be) in ((w0, g0, be0), (w1, g1, be1), (w2, g2, be2)):
        hw = lax.dot_general(w[...], h, (((0,), (0,)), ((), ())),
                             preferred_element_type=jnp.float32)
        mean = lax.dot_general(hw, ones, (((1,), (0,)), ((), ())),
                               preferred_element_type=jnp.float32)
        msq = lax.dot_general(hw * hw, ones, (((1,), (0,)), ((), ())),
                              preferred_element_type=jnp.float32)
        var = msq - mean * mean
        scale = g[...][:, None] * lax.rsqrt(var + 1e-5)
        h = jnp.maximum((hw - mean) * scale + be[...][:, None], 0.0)
    out = (lax.dot_general(fcw[...][:D, :], gmf, (((0,), (0,)), ((), ())),
                           preferred_element_type=jnp.float32)
           + lax.dot_general(fcw[...][D:, :], h, (((0,), (0,)), ((), ())),
                             preferred_element_type=jnp.float32)
           + fcb[...][:, None])
    out_ref[...] = out


def kernel(x, emb_W, W0, b0, g0, beta0, W1, b1, g1, beta1, W2, b2, g2, beta2,
           fcW, fcb):
    flat = emb_W.reshape(15625, 128, 2, 8).transpose(2, 0, 3, 1).reshape(32000000)
    xflat = x.astype(jnp.int32).reshape(128, 128, 2).transpose(0, 2, 1).reshape(2 * BATCH)
    et4 = _make_gather()(flat, xflat)
    et = et4.transpose(0, 2, 1, 3).reshape(NF, BATCH)

    out2d = pl.pallas_call(
        _mlp_body,
        out_shape=jax.ShapeDtypeStruct((1, BATCH), jnp.float32),
    )(et, W0, b0, g0, beta0, W1, b1, g1, beta1, W2, b2, g2, beta2, fcW, fcb)
    return out2d.reshape(BATCH)


# speedup vs baseline: 1.0651x; 1.0651x over previous
import functools

import jax
import jax.numpy as jnp
from jax import lax
from jax.experimental import pallas as pl
from jax.experimental.pallas import tpu as pltpu
from jax.experimental.pallas import tpu_sc as plsc

FIELD0 = 1000000
BATCH = 16384
D = 16
NF = 2 * D

_NW = 32
_COLS_PER_W = BATCH // _NW
_NG = _COLS_PER_W // 128
_CC = _COLS_PER_W // 16

_FOFF = tuple((c // 8) * 16000000 + (c % 8) * 128 for c in range(D))


def _gather_body(tab_hbm, x_hbm, out_hbm, xv, ebuf, rows_v, sem0, sem1):
    wid = lax.axis_index("s") * 2 + lax.axis_index("c")
    base = wid * 2 * _COLS_PER_W
    pltpu.sync_copy(x_hbm.at[pl.ds(base, 2 * _COLS_PER_W)], xv)
    sems = (sem0, sem1)

    def drain_and_flush(g):
        pltpu.make_async_copy(
            out_hbm.at[:, _NG * wid + g], rows_v.at[:, g], sems[g % 2]
        ).wait()
        pltpu.sync_copy(rows_v.at[:, g], out_hbm.at[:, _NG * wid + g])

    for g in range(_NG):
        def build(part, _, g=g):
            ru = xv[pl.ds(256 * g + 16 * part, 16)]
            ri = xv[pl.ds(256 * g + 128 + 16 * part, 16)] + FIELD0
            rpu = lax.shift_right_logical(ru, 7) * 1024 + (ru & 127)
            rpi = lax.shift_right_logical(ri, 7) * 1024 + (ri & 127)
            col = 128 * g + 16 * part
            for c in range(D):
                ebuf[c, pl.ds(col, 16)] = rpu + _FOFF[c]
                ebuf[D + c, pl.ds(col, 16)] = rpi + _FOFF[c]
            return _

        lax.fori_loop(0, 8, build, None)
        for f in range(NF):
            pltpu.async_copy(
                tab_hbm.at[ebuf.at[f, pl.ds(128 * g, 128)]],
                rows_v.at[f // 8, g, f % 8],
                sems[g % 2],
            )
        if g >= 1:
            drain_and_flush(g - 1)
    drain_and_flush(_NG - 1)


@functools.cache
def _make_gather():
    return functools.partial(
        pl.kernel,
        mesh=plsc.VectorSubcoreMesh(core_axis_name="c", subcore_axis_name="s"),
        out_type=jax.ShapeDtypeStruct((NF // 8, BATCH // 128, 8, 128), jnp.float32),
        scratch_types=[
            pltpu.VMEM((2 * _COLS_PER_W,), jnp.int32),
            pltpu.VMEM((NF, _COLS_PER_W), jnp.int32),
            pltpu.VMEM((NF // 8, _NG, 8, 128), jnp.float32),
            pltpu.SemaphoreType.DMA,
            pltpu.SemaphoreType.DMA,
        ],
        compiler_params=pltpu.CompilerParams(use_tc_tiling_on_sc=False),
    )(_gather_body)


def _mlp_body(et_ref, w0, b0, g0, be0, w1, b1, g1, be1, w2, b2, g2, be2,
              fcw, fcb, out_ref):
    et = et_ref[...]
    gmf = et[:D, :] * et[D:, :]
    h = et
    for (w, g, be) in ((w0, g0, be0), (w1, g1, be1), (w2, g2, be2)):
        hw = lax.dot_general(w[...], h, (((0,), (0,)), ((), ())),
                             preferred_element_type=jnp.float32)
        mean = jnp.mean(hw, axis=1, keepdims=True)
        msq = jnp.mean(hw * hw, axis=1, keepdims=True)
        var = msq - mean * mean
        scale = g[...][:, None] * lax.rsqrt(var + 1e-5)
        h = jnp.maximum((hw - mean) * scale + be[...][:, None], 0.0)
    out = (lax.dot_general(fcw[...][:D, :], gmf, (((0,), (0,)), ((), ())),
                           preferred_element_type=jnp.float32)
           + lax.dot_general(fcw[...][D:, :], h, (((0,), (0,)), ((), ())),
                             preferred_element_type=jnp.float32)
           + fcb[...][:, None])
    out_ref[...] = out


def kernel(x, emb_W, W0, b0, g0, beta0, W1, b1, g1, beta1, W2, b2, g2, beta2,
           fcW, fcb):
    flat = emb_W.reshape(15625, 128, 2, 8).transpose(2, 0, 3, 1).reshape(32000000)
    xflat = x.astype(jnp.int32).reshape(128, 128, 2).transpose(0, 2, 1).reshape(2 * BATCH)
    et4 = _make_gather()(flat, xflat)
    et = et4.transpose(0, 2, 1, 3).reshape(NF, BATCH)

    out2d = pl.pallas_call(
        _mlp_body,
        out_shape=jax.ShapeDtypeStruct((1, BATCH), jnp.float32),
    )(et, W0, b0, g0, beta0, W1, b1, g1, beta1, W2, b2, g2, beta2, fcW, fcb)
    return out2d.reshape(BATCH)


# speedup vs baseline: 1.0810x; 1.0149x over previous
import functools

import jax
import jax.numpy as jnp
from jax import lax
from jax.experimental import pallas as pl
from jax.experimental.pallas import tpu as pltpu
from jax.experimental.pallas import tpu_sc as plsc

FIELD0 = 1000000
BATCH = 16384
D = 16
NF = 2 * D

_NW = 32
_COLS_PER_W = BATCH // _NW
_NG = _COLS_PER_W // 128
_CC = _COLS_PER_W // 16

_FOFF = tuple((c // 8) * 16000000 + (c % 8) * 128 for c in range(D))


def _gather_body(tab_hbm, x_hbm, out_hbm, xv, ebuf, rows_v, sem0, sem1):
    wid = lax.axis_index("s") * 2 + lax.axis_index("c")
    base = wid * 2 * _COLS_PER_W
    pltpu.sync_copy(x_hbm.at[pl.ds(base, 2 * _COLS_PER_W)], xv)

    def drain_and_flush(g, sem):
        pltpu.make_async_copy(
            out_hbm.at[:, _NG * wid + g], rows_v.at[:, g], sem
        ).wait()
        pltpu.sync_copy(rows_v.at[:, g], out_hbm.at[:, _NG * wid + g])

    def build(g):
        def part_body(part, _):
            ru = xv[pl.ds(256 * g + 16 * part, 16)]
            ri = xv[pl.ds(256 * g + 128 + 16 * part, 16)] + FIELD0
            rpu = lax.shift_right_logical(ru, 7) * 1024 + (ru & 127)
            rpi = lax.shift_right_logical(ri, 7) * 1024 + (ri & 127)
            col = 128 * g + 16 * part
            for c in range(D):
                ebuf[c, pl.ds(col, 16)] = rpu + _FOFF[c]
                ebuf[D + c, pl.ds(col, 16)] = rpi + _FOFF[c]
            return _

        lax.fori_loop(0, 8, part_body, None)

    def fire(g, sem):
        for f in range(NF):
            pltpu.async_copy(
                tab_hbm.at[ebuf.at[f, pl.ds(128 * g, 128)]],
                rows_v.at[f // 8, g, f % 8],
                sem,
            )

    def pair(i, _):
        ga, gb = 2 * i, 2 * i + 1
        build(ga)
        fire(ga, sem0)
        build(gb)

        @pl.when(i >= 1)
        def _drain_prev():
            drain_and_flush(gb - 2, sem1)

        fire(gb, sem1)
        drain_and_flush(ga, sem0)
        return _

    lax.fori_loop(0, _NG // 2, pair, None)
    drain_and_flush(_NG - 1, sem1)


@functools.cache
def _make_gather():
    return functools.partial(
        pl.kernel,
        mesh=plsc.VectorSubcoreMesh(core_axis_name="c", subcore_axis_name="s"),
        out_type=jax.ShapeDtypeStruct((NF // 8, BATCH // 128, 8, 128), jnp.float32),
        scratch_types=[
            pltpu.VMEM((2 * _COLS_PER_W,), jnp.int32),
            pltpu.VMEM((NF, _COLS_PER_W), jnp.int32),
            pltpu.VMEM((NF // 8, _NG, 8, 128), jnp.float32),
            pltpu.SemaphoreType.DMA,
            pltpu.SemaphoreType.DMA,
        ],
        compiler_params=pltpu.CompilerParams(use_tc_tiling_on_sc=False),
    )(_gather_body)


def _mlp_body(et_ref, w0, b0, g0, be0, w1, b1, g1, be1, w2, b2, g2, be2,
              fcw, fcb, out_ref):
    et = et_ref[...]
    gmf = et[:D, :] * et[D:, :]
    h = et
    for (w, g, be) in ((w0, g0, be0), (w1, g1, be1), (w2, g2, be2)):
        hw = lax.dot_general(w[...], h, (((0,), (0,)), ((), ())),
                             preferred_element_type=jnp.float32)
        mean = jnp.mean(hw, axis=1, keepdims=True)
        msq = jnp.mean(hw * hw, axis=1, keepdims=True)
        var = msq - mean * mean
        scale = g[...][:, None] * lax.rsqrt(var + 1e-5)
        h = jnp.maximum((hw - mean) * scale + be[...][:, None], 0.0)
    out = (lax.dot_general(fcw[...][:D, :], gmf, (((0,), (0,)), ((), ())),
                           preferred_element_type=jnp.float32)
           + lax.dot_general(fcw[...][D:, :], h, (((0,), (0,)), ((), ())),
                             preferred_element_type=jnp.float32)
           + fcb[...][:, None])
    out_ref[...] = out


def kernel(x, emb_W, W0, b0, g0, beta0, W1, b1, g1, beta1, W2, b2, g2, beta2,
           fcW, fcb):
    flat = emb_W.reshape(15625, 128, 2, 8).transpose(2, 0, 3, 1).reshape(32000000)
    xflat = x.astype(jnp.int32).reshape(128, 128, 2).transpose(0, 2, 1).reshape(2 * BATCH)
    et4 = _make_gather()(flat, xflat)
    et = et4.transpose(0, 2, 1, 3).reshape(NF, BATCH)

    out2d = pl.pallas_call(
        _mlp_body,
        out_shape=jax.ShapeDtypeStruct((1, BATCH), jnp.float32),
    )(et, W0, b0, g0, beta0, W1, b1, g1, beta1, W2, b2, g2, beta2, fcW, fcb)
    return out2d.reshape(BATCH)
